# baseline (speedup 1.0000x reference)
import jax
import jax.numpy as jnp
from jax.experimental import pallas as pl

N, D, H, OUT = 10000, 128, 32, 9
ROWS = 1000


def _cell_kernel(x_ref, h_ref, c_ref,
                 wx_ref, wh_ref, bias_ref,
                 wci_ref, wcf_ref, wco_ref,
                 wlin_ref, blin_ref,
                 y_ref, h0_ref, c0_ref):
    x = x_ref[...]
    h = h_ref[...]
    c = c_ref[...]
    g = (jnp.dot(x, wx_ref[...], preferred_element_type=jnp.float32)
         + jnp.dot(h, wh_ref[...], preferred_element_type=jnp.float32)
         + bias_ref[...])
    gi = g[:, 0 * H:1 * H]
    gf = g[:, 1 * H:2 * H]
    gt = g[:, 2 * H:3 * H]
    go = g[:, 3 * H:4 * H]
    i = jax.nn.sigmoid(gi + wci_ref[...] * c)
    f = jax.nn.sigmoid(gf + wcf_ref[...] * c)
    t = jnp.tanh(gt)
    c0 = f * c + i * t
    o = jax.nn.sigmoid(go + wco_ref[...] * c0)
    h0 = o * jnp.tanh(c0)
    c0_ref[...] = c0
    h0_ref[...] = h0
    y_ref[...] = (jnp.dot(jax.nn.relu(h0), wlin_ref[...],
                          preferred_element_type=jnp.float32)
                  + blin_ref[...])


def kernel(x, edge_index, edge_weight, h, c,
           W_xi, b_xi, W_hi, b_hi, w_ci, b_i,
           W_xf, b_xf, W_hf, b_hf, w_cf, b_f,
           W_xc, b_xc, W_hc, b_hc, b_c,
           W_xo, b_xo, W_ho, b_ho, w_co, b_o,
           W_lin, b_lin):
    del edge_index, edge_weight
    Wx = jnp.concatenate([W_xi, W_xf, W_xc, W_xo], axis=1)
    Wh = jnp.concatenate([W_hi, W_hf, W_hc, W_ho], axis=1)
    bias = jnp.concatenate([b_xi + b_hi + b_i[0],
                            b_xf + b_hf + b_f[0],
                            b_xc + b_hc + b_c[0],
                            b_xo + b_ho + b_o[0]], axis=0)[None, :]

    grid = (N // ROWS,)
    row_spec = lambda w: pl.BlockSpec((ROWS, w), lambda i: (i, 0))
    full = lambda a: pl.BlockSpec(a.shape, lambda i: (0,) * a.ndim)

    y, h0, c0 = pl.pallas_call(
        _cell_kernel,
        grid=grid,
        in_specs=[
            row_spec(D),
            row_spec(H),
            row_spec(H),
            full(Wx), full(Wh), full(bias),
            full(w_ci), full(w_cf), full(w_co),
            full(W_lin), full(b_lin[None, :]),
        ],
        out_specs=[row_spec(OUT), row_spec(H), row_spec(H)],
        out_shape=[
            jax.ShapeDtypeStruct((N, OUT), jnp.float32),
            jax.ShapeDtypeStruct((N, H), jnp.float32),
            jax.ShapeDtypeStruct((N, H), jnp.float32),
        ],
    )(x, h, c, Wx, Wh, bias, w_ci, w_cf, w_co, W_lin, b_lin[None, :])
    return (y, h0, c0)


# speedup vs baseline: 1.0700x; 1.0700x over previous
import jax
import jax.numpy as jnp
from jax.experimental import pallas as pl

N, D, H, OUT = 10000, 128, 32, 9
ROWS = 2000


def _dot(a, b_ref):
    return jnp.dot(a, b_ref[...], preferred_element_type=jnp.float32)


def _cell_kernel(x_ref, h_ref, c_ref,
                 wxi_ref, whi_ref, wci_ref, bi_ref,
                 wxf_ref, whf_ref, wcf_ref, bf_ref,
                 wxc_ref, whc_ref, bc_ref,
                 wxo_ref, who_ref, wco_ref, bo_ref,
                 wlin_ref, blin_ref,
                 y_ref, h0_ref, c0_ref):
    x = x_ref[...]
    h = h_ref[...]
    c = c_ref[...]
    i = jax.nn.sigmoid(_dot(x, wxi_ref) + _dot(h, whi_ref)
                       + wci_ref[...] * c + bi_ref[...])
    f = jax.nn.sigmoid(_dot(x, wxf_ref) + _dot(h, whf_ref)
                       + wcf_ref[...] * c + bf_ref[...])
    t = jnp.tanh(_dot(x, wxc_ref) + _dot(h, whc_ref) + bc_ref[...])
    c0 = f * c + i * t
    o = jax.nn.sigmoid(_dot(x, wxo_ref) + _dot(h, who_ref)
                       + wco_ref[...] * c0 + bo_ref[...])
    h0 = o * jnp.tanh(c0)
    c0_ref[...] = c0
    h0_ref[...] = h0
    y_ref[...] = _dot(jax.nn.relu(h0), wlin_ref) + blin_ref[...]


def kernel(x, edge_index, edge_weight, h, c,
           W_xi, b_xi, W_hi, b_hi, w_ci, b_i,
           W_xf, b_xf, W_hf, b_hf, w_cf, b_f,
           W_xc, b_xc, W_hc, b_hc, b_c,
           W_xo, b_xo, W_ho, b_ho, w_co, b_o,
           W_lin, b_lin):
    del edge_index, edge_weight

    bi = (b_xi + b_hi + b_i[0])[None, :]
    bf = (b_xf + b_hf + b_f[0])[None, :]
    bc = (b_xc + b_hc + b_c[0])[None, :]
    bo = (b_xo + b_ho + b_o[0])[None, :]

    grid = (N // ROWS,)
    row_spec = lambda w: pl.BlockSpec((ROWS, w), lambda i: (i, 0))
    full = lambda a: pl.BlockSpec(a.shape, lambda i: (0,) * a.ndim)

    ins = (x, h, c,
           W_xi, W_hi, w_ci, bi,
           W_xf, W_hf, w_cf, bf,
           W_xc, W_hc, bc,
           W_xo, W_ho, w_co, bo,
           W_lin, b_lin[None, :])

    y, h0, c0 = pl.pallas_call(
        _cell_kernel,
        grid=grid,
        in_specs=[row_spec(D), row_spec(H), row_spec(H)]
                 + [full(a) for a in ins[3:]],
        out_specs=[row_spec(OUT), row_spec(H), row_spec(H)],
        out_shape=[
            jax.ShapeDtypeStruct((N, OUT), jnp.float32),
            jax.ShapeDtypeStruct((N, H), jnp.float32),
            jax.ShapeDtypeStruct((N, H), jnp.float32),
        ],
    )(*ins)
    return (y, h0, c0)


# speedup vs baseline: 1.1277x; 1.0539x over previous
import jax
import jax.numpy as jnp
from jax.experimental import pallas as pl

N, D, H, OUT = 10000, 128, 32, 9
ROWS = 2000


def _dot(a, b_ref):
    return jnp.dot(a, b_ref[...], preferred_element_type=jnp.float32)


def _cell_kernel(x_ref, h_ref, c_ref,
                 wxi_ref, whi_ref, wci_ref, bxi_ref, bhi_ref, bi_ref,
                 wxf_ref, whf_ref, wcf_ref, bxf_ref, bhf_ref, bf_ref,
                 wxc_ref, whc_ref, bxc_ref, bhc_ref, bc_ref,
                 wxo_ref, who_ref, wco_ref, bxo_ref, bho_ref, bo_ref,
                 wlin_ref, blin_ref,
                 y_ref, h0_ref, c0_ref):
    x = x_ref[...]
    h = h_ref[...]
    c = c_ref[...]
    bi = bxi_ref[...] + bhi_ref[...] + bi_ref[...]
    bf = bxf_ref[...] + bhf_ref[...] + bf_ref[...]
    bc = bxc_ref[...] + bhc_ref[...] + bc_ref[...]
    bo = bxo_ref[...] + bho_ref[...] + bo_ref[...]
    i = jax.nn.sigmoid(_dot(x, wxi_ref) + _dot(h, whi_ref)
                       + wci_ref[...] * c + bi)
    f = jax.nn.sigmoid(_dot(x, wxf_ref) + _dot(h, whf_ref)
                       + wcf_ref[...] * c + bf)
    t = jnp.tanh(_dot(x, wxc_ref) + _dot(h, whc_ref) + bc)
    c0 = f * c + i * t
    o = jax.nn.sigmoid(_dot(x, wxo_ref) + _dot(h, who_ref)
                       + wco_ref[...] * c0 + bo)
    h0 = o * jnp.tanh(c0)
    c0_ref[...] = c0
    h0_ref[...] = h0
    y_ref[...] = _dot(jax.nn.relu(h0), wlin_ref) + blin_ref[...]


def kernel(x, edge_index, edge_weight, h, c,
           W_xi, b_xi, W_hi, b_hi, w_ci, b_i,
           W_xf, b_xf, W_hf, b_hf, w_cf, b_f,
           W_xc, b_xc, W_hc, b_hc, b_c,
           W_xo, b_xo, W_ho, b_ho, w_co, b_o,
           W_lin, b_lin):
    del edge_index, edge_weight

    grid = (N // ROWS,)
    row_spec = lambda w: pl.BlockSpec((ROWS, w), lambda i: (i, 0))
    full = lambda a: pl.BlockSpec(a.shape, lambda i: (0,) * a.ndim)

    r = lambda b: b.reshape(1, -1)
    ins = (x, h, c,
           W_xi, W_hi, w_ci, r(b_xi), r(b_hi), b_i,
           W_xf, W_hf, w_cf, r(b_xf), r(b_hf), b_f,
           W_xc, W_hc, r(b_xc), r(b_hc), b_c,
           W_xo, W_ho, w_co, r(b_xo), r(b_ho), b_o,
           W_lin, r(b_lin))

    y, h0, c0 = pl.pallas_call(
        _cell_kernel,
        grid=grid,
        in_specs=[row_spec(D), row_spec(H), row_spec(H)]
                 + [full(a) for a in ins[3:]],
        out_specs=[row_spec(OUT), row_spec(H), row_spec(H)],
        out_shape=[
            jax.ShapeDtypeStruct((N, OUT), jnp.float32),
            jax.ShapeDtypeStruct((N, H), jnp.float32),
            jax.ShapeDtypeStruct((N, H), jnp.float32),
        ],
    )(*ins)
    return (y, h0, c0)
